# Initial kernel scaffold; baseline (speedup 1.0000x reference)
import jax
import jax.numpy as jnp
from jax.experimental import pallas as pl


def kernel(x, edge_index, W0, b0, W1, b1, W2, b2, W3, b3, W4, b4):
    raise NotImplementedError("write your pallas kernel here")


# speedup vs baseline: 9.7618x; 9.7618x over previous
import functools

import jax
import jax.numpy as jnp
from jax import lax
from jax.experimental import pallas as pl
from jax.experimental.pallas import tpu as pltpu
from jax.experimental.pallas import tpu_sc as plsc

NC = 2
NS = 16
NW = NC * NS
CH = 128
D = 128


def _mesh():
    return plsc.VectorSubcoreMesh(
        core_axis_name="c", subcore_axis_name="s", num_cores=NC, num_subcores=NS
    )


@functools.lru_cache(maxsize=None)
def _make_scatter(n_acc, cpw):
    zpt = n_acc // NS
    zc = zpt // CH

    @functools.partial(
        pl.kernel,
        out_type=jax.ShapeDtypeStruct((NC, n_acc, D), jnp.float32),
        mesh=_mesh(),
        scratch_types=[
            pltpu.VMEM((cpw, CH), jnp.int32),
            pltpu.VMEM((cpw, CH), jnp.int32),
            pltpu.VMEM((CH, D), jnp.float32),
            pltpu.VMEM_SHARED((n_acc, D), jnp.float32),
            pltpu.SemaphoreType.DMA,
        ],
    )
    def k(y_hbm, src_hbm, dst_hbm, out_hbm, sbuf, dbuf, rows, acc, sem):
        cid = lax.axis_index("c")
        tid = lax.axis_index("s")
        wid = tid * NC + cid

        def zf(i, _):
            for kk in range(D // 16):
                rows[i, pl.ds(kk * 16, 16)] = jnp.zeros((16,), jnp.float32)
            return 0

        lax.fori_loop(0, CH, zf, 0)
        for m in range(zc):
            pltpu.sync_copy(rows, acc.at[pl.ds(tid * zpt + m * CH, CH)])

        pltpu.sync_copy(src_hbm.at[wid], sbuf)
        pltpu.sync_copy(dst_hbm.at[wid], dbuf)
        plsc.subcore_barrier()

        def body(j, _):
            pltpu.async_copy(y_hbm.at[sbuf.at[j]], rows, sem).wait()
            pltpu.sync_copy(rows, acc.at[dbuf.at[j]], add=True)
            return 0

        lax.fori_loop(0, cpw, body, 0)
        plsc.subcore_barrier()

        for m in range(zc):
            r0 = tid * zpt + m * CH
            pltpu.sync_copy(acc.at[pl.ds(r0, CH)], rows)
            pltpu.sync_copy(rows, out_hbm.at[cid, pl.ds(r0, CH)])

    return k


@functools.lru_cache(maxsize=None)
def _make_deg(n_acc, cpw):
    zpt = n_acc // NS

    @functools.partial(
        pl.kernel,
        out_type=jax.ShapeDtypeStruct((NC, n_acc), jnp.float32),
        mesh=_mesh(),
        scratch_types=[
            pltpu.VMEM((cpw, CH), jnp.int32),
            pltpu.VMEM((CH,), jnp.float32),
            pltpu.VMEM((zpt,), jnp.float32),
            pltpu.VMEM_SHARED((n_acc,), jnp.float32),
        ],
    )
    def k(dst_hbm, out_hbm, dbuf, ones_v, zbuf, accd):
        cid = lax.axis_index("c")
        tid = lax.axis_index("s")
        wid = tid * NC + cid

        def zf(i, _):
            zbuf[pl.ds(i * 16, 16)] = jnp.zeros((16,), jnp.float32)
            return 0

        lax.fori_loop(0, zpt // 16, zf, 0)
        pltpu.sync_copy(zbuf, accd.at[pl.ds(tid * zpt, zpt)])
        for kk in range(CH // 16):
            ones_v[pl.ds(kk * 16, 16)] = jnp.ones((16,), jnp.float32)
        pltpu.sync_copy(dst_hbm.at[wid], dbuf)
        plsc.subcore_barrier()

        def body(j, _):
            pltpu.sync_copy(ones_v, accd.at[dbuf.at[j]], add=True)
            return 0

        lax.fori_loop(0, cpw, body, 0)
        plsc.subcore_barrier()
        pltpu.sync_copy(accd.at[pl.ds(tid * zpt, zpt)], zbuf)
        pltpu.sync_copy(zbuf, out_hbm.at[cid, pl.ds(tid * zpt, zpt)])

    return k


def _tc_pre(x, degt, w0):
    n = x.shape[0]
    r = 2000

    def body(x_ref, dg_ref, w_ref, y_ref, dis_ref):
        dis = lax.rsqrt(dg_ref[:, 0:1] + dg_ref[:, 1:2] + 1.0)
        xw = jnp.dot(x_ref[...], w_ref[...], preferred_element_type=jnp.float32)
        y_ref[...] = xw * dis
        dis_ref[...] = dis

    return pl.pallas_call(
        body,
        grid=(n // r,),
        in_specs=[
            pl.BlockSpec((r, D), lambda i: (i, 0)),
            pl.BlockSpec((r, 2), lambda i: (i, 0)),
            pl.BlockSpec((D, D), lambda i: (0, 0)),
        ],
        out_specs=[
            pl.BlockSpec((r, D), lambda i: (i, 0)),
            pl.BlockSpec((r, 1), lambda i: (i, 0)),
        ],
        out_shape=[
            jax.ShapeDtypeStruct((n, D), jnp.float32),
            jax.ShapeDtypeStruct((n, 1), jnp.float32),
        ],
    )(x, degt, w0)


def _tc_mid(p0, p1, y, dis, b, w_next):
    n = y.shape[0]
    r = 2000

    def body(p0_ref, p1_ref, y_ref, dis_ref, b_ref, w_ref, o_ref):
        dis = dis_ref[...]
        z = (p0_ref[...] + p1_ref[...] + y_ref[...]) * dis + b_ref[...]
        h = jnp.maximum(z, 0.0)
        o_ref[...] = (
            jnp.dot(h, w_ref[...], preferred_element_type=jnp.float32) * dis
        )

    return pl.pallas_call(
        body,
        grid=(n // r,),
        in_specs=[
            pl.BlockSpec((r, D), lambda i: (i, 0)),
            pl.BlockSpec((r, D), lambda i: (i, 0)),
            pl.BlockSpec((r, D), lambda i: (i, 0)),
            pl.BlockSpec((r, 1), lambda i: (i, 0)),
            pl.BlockSpec((1, D), lambda i: (0, 0)),
            pl.BlockSpec((D, D), lambda i: (0, 0)),
        ],
        out_specs=pl.BlockSpec((r, D), lambda i: (i, 0)),
        out_shape=jax.ShapeDtypeStruct((n, D), jnp.float32),
    )(p0, p1, y, dis, b, w_next)


def _tc_fin(p0, p1, y, dis, b):
    n = y.shape[0]
    r = 2000

    def body(p0_ref, p1_ref, y_ref, dis_ref, b_ref, o_ref):
        o_ref[...] = (
            (p0_ref[...] + p1_ref[...] + y_ref[...]) * dis_ref[...] + b_ref[...]
        )

    return pl.pallas_call(
        body,
        grid=(n // r,),
        in_specs=[
            pl.BlockSpec((r, D), lambda i: (i, 0)),
            pl.BlockSpec((r, D), lambda i: (i, 0)),
            pl.BlockSpec((r, D), lambda i: (i, 0)),
            pl.BlockSpec((r, 1), lambda i: (i, 0)),
            pl.BlockSpec((1, D), lambda i: (0, 0)),
        ],
        out_specs=pl.BlockSpec((r, D), lambda i: (i, 0)),
        out_shape=jax.ShapeDtypeStruct((n, D), jnp.float32),
    )(p0, p1, y, dis, b)


def kernel(x, edge_index, W0, b0, W1, b1, W2, b2, W3, b3, W4, b4):
    n = x.shape[0]
    e = edge_index.shape[1]
    n_acc = -(-(n + 1) // (NS * CH)) * (NS * CH)
    cpw = -(-e // (NW * CH))
    e_pad = cpw * CH * NW

    src = edge_index[0]
    dst = edge_index[1]
    pad = e_pad - e
    srcp = jnp.concatenate([src, jnp.zeros((pad,), src.dtype)]).reshape(NW, cpw, CH)
    dstp = jnp.concatenate([dst, jnp.full((pad,), n, dst.dtype)]).reshape(NW, cpw, CH)

    degp = _make_deg(n_acc, cpw)(dstp)
    degt = jnp.stack([degp[0, :n], degp[1, :n]], axis=1)
    y, dis = _tc_pre(x, degt, W0)

    scatter = _make_scatter(n_acc, cpw)
    ws = [W1, W2, W3, W4]
    bs = [b0, b1, b2, b3, b4]
    for i in range(5):
        p = scatter(y, srcp, dstp)
        p0 = p[0, :n]
        p1 = p[1, :n]
        if i < 4:
            y = _tc_mid(p0, p1, y, dis, bs[i].reshape(1, D), ws[i])
        else:
            out = _tc_fin(p0, p1, y, dis, bs[i].reshape(1, D))
    return out
